# baseline (speedup 1.0000x reference)
import jax
import jax.numpy as jnp
from jax import lax
from jax.experimental import pallas as pl
from jax.experimental.pallas import tpu as pltpu
from jax.experimental.pallas import tpu_sc as plsc

NUM_HEADS = 8
N_GLOBAL = 3
SEQ_LEN = 67
TOTAL = NUM_HEADS * SEQ_LEN * SEQ_LEN
NC, NS, LANES = 2, 16, 16
NW = NC * NS
CHUNK = 1136
VECS = CHUNK // LANES
LAST_CHUNK = TOTAL - (NW - 1) * CHUNK

ROW_FE = 9
ROW_DB = 17
ROW_AB = 18
ROW_KB = 19


def _sc_body(tab_h, gb_h, out_h, tab_v, gb_v, chunk_v, sem):
    wid = lax.axis_index("s") * NC + lax.axis_index("c")
    c1 = pltpu.async_copy(tab_h, tab_v, sem)
    c2 = pltpu.async_copy(gb_h, gb_v, sem)
    c1.wait()
    c2.wait()

    base = wid * CHUNK
    p0 = base + lax.iota(jnp.int32, LANES)
    zeros = jnp.zeros((LANES,), jnp.float32)
    rdb = jnp.full((LANES,), ROW_DB, jnp.int32)
    rab = jnp.full((LANES,), ROW_AB, jnp.int32)
    rkb = jnp.full((LANES,), ROW_KB, jnp.int32)

    @plsc.parallel_loop(0, VECS)
    def _(v):
        p = jnp.minimum(p0 + (v << 4), TOTAL - 1)
        h = (p * 7475) >> 25
        rem = p - h * 4489
        i = (rem * 3913) >> 18
        j = rem - i * 67

        si = i - N_GLOBAL
        sj = j - N_GLOBAL
        ri = si >> 3
        fi = si & 7
        rj = sj >> 3
        fj = sj & 7
        dr = jnp.abs(ri - rj)
        df = jnp.abs(fi - fj)
        v_sq = (plsc.load_gather(tab_v, [dr, h])
                + plsc.load_gather(tab_v, [ROW_FE + df, h]))
        v_sq = v_sq + jnp.where(ri - fi == rj - fj,
                                plsc.load_gather(tab_v, [rdb, h]), zeros)
        v_sq = v_sq + jnp.where(ri + fi == rj + fj,
                                plsc.load_gather(tab_v, [rab, h]), zeros)
        v_sq = v_sq + jnp.where(dr * df == 2,
                                plsc.load_gather(tab_v, [rkb, h]), zeros)

        is_top = i < N_GLOBAL
        gmid = jnp.where(is_top, i, jnp.minimum(j, N_GLOBAL - 1))
        glast = jnp.where(is_top, j, i)
        v_glob = plsc.load_gather(gb_v, [h * N_GLOBAL + gmid, glast])

        in_sq = (i >= N_GLOBAL) & (j >= N_GLOBAL)
        chunk_v[pl.ds(v * LANES, LANES)] = jnp.where(in_sq, v_sq, v_glob)

    @pl.when(wid < NW - 1)
    def _():
        pltpu.sync_copy(chunk_v, out_h.at[pl.ds(base, CHUNK)])

    @pl.when(wid == NW - 1)
    def _():
        pltpu.sync_copy(chunk_v.at[pl.ds(0, LAST_CHUNK)],
                        out_h.at[pl.ds((NW - 1) * CHUNK, LAST_CHUNK)])


def kernel(rank_embed, file_embed, diag_bias, antidiag_bias, knight_bias,
           global_bias, rank_diff, file_diff, same_diag, same_antidiag,
           knight_reach):
    tab = jnp.concatenate([
        rank_embed, jnp.zeros((1, NUM_HEADS), rank_embed.dtype),
        file_embed,
        diag_bias[None], antidiag_bias[None], knight_bias[None],
    ])
    gb24 = global_bias.reshape(NUM_HEADS * N_GLOBAL, SEQ_LEN)
    flat = pl.kernel(
        _sc_body,
        out_type=jax.ShapeDtypeStruct((TOTAL,), jnp.float32),
        mesh=plsc.VectorSubcoreMesh(core_axis_name="c", subcore_axis_name="s",
                                    num_cores=NC, num_subcores=NS),
        compiler_params=pltpu.CompilerParams(needs_layout_passes=False),
        scratch_types=[
            pltpu.VMEM((20, NUM_HEADS), jnp.float32),
            pltpu.VMEM((NUM_HEADS * N_GLOBAL, SEQ_LEN), jnp.float32),
            pltpu.VMEM((CHUNK,), jnp.float32),
            pltpu.SemaphoreType.DMA,
        ],
    )(tab, gb24)
    return flat.reshape(1, NUM_HEADS, SEQ_LEN, SEQ_LEN)


# speedup vs baseline: 1.2107x; 1.2107x over previous
import jax
import jax.numpy as jnp
from jax import lax
from jax.experimental import pallas as pl
from jax.experimental.pallas import tpu as pltpu
from jax.experimental.pallas import tpu_sc as plsc

NUM_HEADS = 8
N_GLOBAL = 3
SEQ_LEN = 67
TOTAL = NUM_HEADS * SEQ_LEN * SEQ_LEN
NC, NS, LANES = 2, 16, 16
NW = NC * NS
NR = 16
NR_LAST = SEQ_LEN - 3 * NR
VECS = (NR_LAST * SEQ_LEN + LANES - 1) // LANES

OFF_FE = 72
OFF_DB = 136
OFF_AB = 144
OFF_KB = 152
OFF_GB = 160
TAB_LEN = 1792


def _sc_body(tab_h, out_h, tab_v, chunk_v):
    wid = lax.axis_index("s") * NC + lax.axis_index("c")
    pltpu.sync_copy(tab_h, tab_v)

    h = wid >> 2
    q = wid & 3
    r0 = q * NR
    nrows = jnp.where(q < 3, NR, NR_LAST)
    start = h * 4489 + r0 * 67
    last = start + nrows * 67 - 1
    p0 = start + lax.iota(jnp.int32, LANES)
    zeros = jnp.zeros((LANES,), jnp.float32)
    zi = jnp.zeros((LANES,), jnp.int32)
    idx_db = zi + (OFF_DB + h)
    idx_ab = zi + (OFF_AB + h)
    idx_kb = zi + (OFF_KB + h)

    @plsc.parallel_loop(0, VECS)
    def _(v):
        p = jnp.minimum(p0 + (v << 4), last)
        rem = p - h * 4489
        i = (rem * 3913) >> 18
        j = rem - i * 67

        si = i - N_GLOBAL
        sj = j - N_GLOBAL
        ri = si >> 3
        fi = si & 7
        rj = sj >> 3
        fj = sj & 7
        dr = jnp.abs(ri - rj)
        df = jnp.abs(fi - fj)
        v_sq = (plsc.load_gather(tab_v, [(dr << 3) + h])
                + plsc.load_gather(tab_v, [(df << 3) + (OFF_FE + h)]))
        v_sq = v_sq + jnp.where(ri - fi == rj - fj,
                                plsc.load_gather(tab_v, [idx_db]), zeros)
        v_sq = v_sq + jnp.where(ri + fi == rj + fj,
                                plsc.load_gather(tab_v, [idx_ab]), zeros)
        v_sq = v_sq + jnp.where(dr * df == 2,
                                plsc.load_gather(tab_v, [idx_kb]), zeros)

        is_top = i < N_GLOBAL
        gmid = jnp.where(is_top, i, jnp.minimum(j, N_GLOBAL - 1))
        glast = jnp.where(is_top, j, i)
        v_glob = plsc.load_gather(
            tab_v, [h * 201 + gmid * 67 + (OFF_GB + glast)])

        in_sq = (i >= N_GLOBAL) & (j >= N_GLOBAL)
        plsc.store_scatter(chunk_v, [i - r0, j],
                           jnp.where(in_sq, v_sq, v_glob))

    @pl.when(q < 3)
    def _():
        pltpu.sync_copy(chunk_v.at[pl.ds(0, NR), :],
                        out_h.at[h, pl.ds(r0, NR)])

    @pl.when(q == 3)
    def _():
        pltpu.sync_copy(chunk_v, out_h.at[h, pl.ds(3 * NR, NR_LAST)])


def kernel(rank_embed, file_embed, diag_bias, antidiag_bias, knight_bias,
           global_bias, rank_diff, file_diff, same_diag, same_antidiag,
           knight_reach):
    z8 = jnp.zeros((8,), rank_embed.dtype)
    tab = jnp.concatenate([
        rank_embed.reshape(-1), z8,
        file_embed.reshape(-1),
        diag_bias, antidiag_bias, knight_bias,
        global_bias.reshape(-1),
        jnp.zeros((TAB_LEN - OFF_GB - NUM_HEADS * N_GLOBAL * SEQ_LEN,),
                  rank_embed.dtype),
    ])
    out = pl.kernel(
        _sc_body,
        out_type=jax.ShapeDtypeStruct((NUM_HEADS, SEQ_LEN, SEQ_LEN),
                                      jnp.float32),
        mesh=plsc.VectorSubcoreMesh(core_axis_name="c", subcore_axis_name="s",
                                    num_cores=NC, num_subcores=NS),
        compiler_params=pltpu.CompilerParams(needs_layout_passes=False),
        scratch_types=[
            pltpu.VMEM((TAB_LEN,), jnp.float32),
            pltpu.VMEM((NR_LAST, SEQ_LEN), jnp.float32),
        ],
    )(tab)
    return out[None]
